# Initial kernel scaffold; baseline (speedup 1.0000x reference)
import jax
import jax.numpy as jnp
from jax.experimental import pallas as pl


def kernel(x, edge_index, batch, params):
    raise NotImplementedError("write your pallas kernel here")


# speedup vs baseline: 7.3570x; 7.3570x over previous
import functools

import jax
import jax.numpy as jnp
from jax import lax
from jax.experimental import pallas as pl
from jax.experimental.pallas import tpu as pltpu
from jax.experimental.pallas import tpu_sc as plsc

_N = 10000
_E = 320000
_H = 128
_NG = 128
_LAT = 64

_CH = 125
_NCH = _E // _CH

_info = plsc.get_sparse_core_info()
_NC = _info.num_cores
_NS = _info.num_subcores
_NW = _NC * _NS
_CPW = _NCH // _NW
_RPT = 624
_RPT_LAST = _N - 15 * _RPT

_sc_mesh = plsc.VectorSubcoreMesh(core_axis_name="c", subcore_axis_name="s")


@functools.partial(
    pl.kernel,
    mesh=_sc_mesh,
    out_type=jax.ShapeDtypeStruct((_NC, _N, _H), jnp.float32),
    scratch_types=[
        pltpu.VMEM((_CPW, _CH), jnp.int32),
        pltpu.VMEM((_CPW, _CH), jnp.int32),
        pltpu.VMEM((_CH, _H), jnp.float32),
        pltpu.VMEM_SHARED((_N, _H), jnp.float32),
        pltpu.SemaphoreType.DMA,
    ],
)
def _sc_segment_sum(h_hbm, src_hbm, dst_hbm, zeros_hbm, out_hbm,
                    src_v, dst_v, rows_v, acc, sem):
    c = lax.axis_index("c")
    s = lax.axis_index("s")
    wid = c * _NS + s

    @pl.when(s < _NS - 1)
    def _():
        pltpu.sync_copy(zeros_hbm.at[pl.ds(0, _RPT)],
                        acc.at[pl.ds(s * _RPT, _RPT)])

    @pl.when(s == _NS - 1)
    def _():
        pltpu.sync_copy(zeros_hbm, acc.at[pl.ds(15 * _RPT, _RPT_LAST)])
    pltpu.sync_copy(src_hbm.at[pl.ds(wid * _CPW, _CPW)], src_v)
    pltpu.sync_copy(dst_hbm.at[pl.ds(wid * _CPW, _CPW)], dst_v)
    plsc.subcore_barrier()

    def body(j, carry):
        pltpu.async_copy(h_hbm.at[src_v.at[j]], rows_v, sem).wait()
        pltpu.sync_copy(rows_v, acc.at[dst_v.at[j]], add=True)
        return carry

    lax.fori_loop(0, _CPW, body, 0)

    plsc.subcore_barrier()

    @pl.when(s < _NS - 1)
    def _():
        pltpu.sync_copy(acc.at[pl.ds(s * _RPT, _RPT)],
                        out_hbm.at[c, pl.ds(s * _RPT, _RPT)])

    @pl.when(s == _NS - 1)
    def _():
        pltpu.sync_copy(acc.at[pl.ds(15 * _RPT, _RPT_LAST)],
                        out_hbm.at[c, pl.ds(15 * _RPT, _RPT_LAST)])


def _lrelu(x):
    return jnp.where(x >= 0, x, 0.2 * x)


def _tc_layer_body(h_ref, a0_ref, a1_ref, w1_ref, b1_ref, g_ref, be_ref,
                   w2_ref, b2_ref, o_ref):
    m = h_ref[...] + a0_ref[...] + a1_ref[...]
    t = jnp.dot(m, w1_ref[...], preferred_element_type=jnp.float32) + b1_ref[...]
    t = _lrelu(t)
    mu = jnp.mean(t, axis=0, keepdims=True)
    var = jnp.mean((t - mu) * (t - mu), axis=0, keepdims=True)
    t = (t - mu) / jnp.sqrt(var + 1e-5) * g_ref[...] + be_ref[...]
    u = jnp.dot(t, w2_ref[...], preferred_element_type=jnp.float32) + b2_ref[...]
    u = _lrelu(_lrelu(u))
    o_ref[...] = u + h_ref[...]


_tc_layer = pl.pallas_call(
    _tc_layer_body,
    out_shape=jax.ShapeDtypeStruct((_N, _H), jnp.float32),
)


def _tc_final_body(h_ref, b_ref, g_ref, be_ref, w_ref, bb_ref, o_ref):
    gid = lax.broadcasted_iota(jnp.int32, (_NG, _N), 0)
    onehot = (b_ref[...] == gid).astype(jnp.float32)
    pooled = jnp.dot(onehot, h_ref[...], preferred_element_type=jnp.float32)
    mu = jnp.mean(pooled, axis=0, keepdims=True)
    var = jnp.mean((pooled - mu) * (pooled - mu), axis=0, keepdims=True)
    p = (pooled - mu) / jnp.sqrt(var + 1e-5) * g_ref[...] + be_ref[...]
    o_ref[...] = jnp.dot(p, w_ref[...], preferred_element_type=jnp.float32) + bb_ref[...]


_tc_final = pl.pallas_call(
    _tc_final_body,
    out_shape=jax.ShapeDtypeStruct((_NG, _LAT), jnp.float32),
)


def kernel(x, edge_index, batch, params):
    src = edge_index[0].reshape(_NCH, _CH)
    dst = edge_index[1].reshape(_NCH, _CH)
    zeros = jnp.zeros((_RPT_LAST, _H), jnp.float32)

    h = x
    for lp in params["layers"]:
        agg = _sc_segment_sum(h, src, dst, zeros)
        h = _tc_layer(h, agg[0], agg[1],
                      lp["W1"], lp["b1"].reshape(1, _H),
                      lp["g"].reshape(1, _H), lp["be"].reshape(1, _H),
                      lp["W2"], lp["b2"].reshape(1, _H))
    out = _tc_final(h, batch.reshape(1, _N),
                    params["bn_g"].reshape(1, _H), params["bn_b"].reshape(1, _H),
                    params["fcW"], params["fcb"].reshape(1, _LAT))
    return out
